# baseline (speedup 1.0000x reference)
import functools

import jax
import jax.numpy as jnp
from jax import lax
from jax.experimental import pallas as pl
from jax.experimental.pallas import tpu as pltpu

BATCH = 16384
D = 64
TCH = 512


def _tc_body(uidx_s, iidx_s, b_s, ut_any, it_any, w_v, out_v, ubuf, ibuf, sem):
    n_chunks = BATCH // TCH
    w = w_v[...].reshape(1, D)
    b = b_s[0]

    def fire(c, buf_u, buf_i):
        cb = c * TCH

        def fire_row(r, carry):
            pltpu.make_async_copy(ut_any.at[uidx_s[cb + r]],
                                  buf_u.at[r], sem).start()
            pltpu.make_async_copy(it_any.at[iidx_s[cb + r]],
                                  buf_i.at[r], sem).start()
            return carry

        lax.fori_loop(0, TCH, fire_row, 0)

    def drain(c, buf_u, buf_i):
        cb = c * TCH

        def drain_row(r, carry):
            pltpu.make_async_copy(ut_any.at[uidx_s[cb + r]],
                                  buf_u.at[r], sem).wait()
            pltpu.make_async_copy(it_any.at[iidx_s[cb + r]],
                                  buf_i.at[r], sem).wait()
            return carry

        lax.fori_loop(0, TCH, drain_row, 0)

    def chunk(c, carry):
        fire(c, ubuf, ibuf)
        drain(c, ubuf, ibuf)
        p = ubuf[...] * ibuf[...] * w
        s = jnp.sum(p, axis=1) + b
        out_v[pl.ds(c * TCH, TCH)] = 1.0 / (1.0 + jnp.exp(-s))
        return carry

    lax.fori_loop(0, n_chunks, chunk, 0)


@jax.jit
def _tc_gmf(user_indices, item_indices, user_table, item_table, W, b):
    return pl.pallas_call(
        _tc_body,
        out_shape=jax.ShapeDtypeStruct((BATCH,), jnp.float32),
        in_specs=[
            pl.BlockSpec(memory_space=pltpu.SMEM),
            pl.BlockSpec(memory_space=pltpu.SMEM),
            pl.BlockSpec(memory_space=pltpu.SMEM),
            pl.BlockSpec(memory_space=pl.ANY),
            pl.BlockSpec(memory_space=pl.ANY),
            pl.BlockSpec(memory_space=pltpu.VMEM),
        ],
        out_specs=pl.BlockSpec(memory_space=pltpu.VMEM),
        scratch_shapes=[
            pltpu.VMEM((TCH, D), jnp.float32),
            pltpu.VMEM((TCH, D), jnp.float32),
            pltpu.SemaphoreType.DMA,
        ],
    )(user_indices, item_indices, b, user_table, item_table, W)


def kernel(user_indices, item_indices, user_table, item_table, W, b):
    return _tc_gmf(user_indices.astype(jnp.int32),
                   item_indices.astype(jnp.int32),
                   user_table, item_table, W.reshape(-1), b)


# speedup vs baseline: 1.0737x; 1.0737x over previous
import functools

import jax
import jax.numpy as jnp
from jax import lax
from jax.experimental import pallas as pl
from jax.experimental.pallas import tpu as pltpu

BATCH = 16384
D = 64
TCH = 512


def _tc_body(uidx_s, iidx_s, b_s, ut_any, it_any, w_v, out_v, ubuf, ibuf, sem):
    n_chunks = BATCH // TCH
    w = w_v[...].reshape(1, D)
    b = b_s[0]

    def fire(c, buf_u, buf_i):
        cb = c * TCH

        def fire_row(r, carry):
            pltpu.make_async_copy(ut_any.at[uidx_s[cb + r]],
                                  buf_u.at[r], sem).start()
            pltpu.make_async_copy(it_any.at[iidx_s[cb + r]],
                                  buf_i.at[r], sem).start()
            return carry

        lax.fori_loop(0, TCH, fire_row, 0, unroll=8)

    def drain(c, buf_u, buf_i):
        del c
        pltpu.make_async_copy(ut_any.at[pl.ds(0, TCH)], buf_u, sem).wait()
        pltpu.make_async_copy(it_any.at[pl.ds(0, TCH)], buf_i, sem).wait()

    def chunk(c, carry):
        fire(c, ubuf, ibuf)
        drain(c, ubuf, ibuf)
        p = ubuf[...] * ibuf[...] * w
        s = jnp.sum(p, axis=1) + b
        out_v[pl.ds(c * TCH, TCH)] = 1.0 / (1.0 + jnp.exp(-s))
        return carry

    lax.fori_loop(0, n_chunks, chunk, 0)


@jax.jit
def _tc_gmf(user_indices, item_indices, user_table, item_table, W, b):
    return pl.pallas_call(
        _tc_body,
        out_shape=jax.ShapeDtypeStruct((BATCH,), jnp.float32),
        in_specs=[
            pl.BlockSpec(memory_space=pltpu.SMEM),
            pl.BlockSpec(memory_space=pltpu.SMEM),
            pl.BlockSpec(memory_space=pltpu.SMEM),
            pl.BlockSpec(memory_space=pl.ANY),
            pl.BlockSpec(memory_space=pl.ANY),
            pl.BlockSpec(memory_space=pltpu.VMEM),
        ],
        out_specs=pl.BlockSpec(memory_space=pltpu.VMEM),
        scratch_shapes=[
            pltpu.VMEM((TCH, D), jnp.float32),
            pltpu.VMEM((TCH, D), jnp.float32),
            pltpu.SemaphoreType.DMA,
        ],
    )(user_indices, item_indices, b, user_table, item_table, W)


def kernel(user_indices, item_indices, user_table, item_table, W, b):
    return _tc_gmf(user_indices.astype(jnp.int32),
                   item_indices.astype(jnp.int32),
                   user_table, item_table, W.reshape(-1), b)


# speedup vs baseline: 1.1715x; 1.0911x over previous
import jax
import jax.numpy as jnp
from jax import lax
from jax.experimental import pallas as pl
from jax.experimental.pallas import tpu as pltpu
from jax.experimental.pallas import tpu_sc as plsc

BATCH = 16384
D = 64
LANES = 16
NC = 2
NS = 16
NW = NC * NS

SC_ROWS = 8704
B_PER_W = SC_ROWS // NW
CH = 256
K = 16

TC_ROWS = BATCH - SC_ROWS
TCH = 512


def _sc_body(ut_hbm, it_hbm, uidx_hbm, iidx_hbm, wb_hbm, out_hbm,
             uidx_v, iidx_v, urows_v, irows_v, wb_v, out_v, scan_v, sem):
    wid = lax.axis_index("s") * NC + lax.axis_index("c")
    base = wid * B_PER_W

    pltpu.sync_copy(uidx_hbm.at[pl.ds(base, B_PER_W)], uidx_v)
    pltpu.sync_copy(iidx_hbm.at[pl.ds(base, B_PER_W)], iidx_v)
    pltpu.sync_copy(wb_hbm, wb_v)

    w0 = wb_v[pl.ds(0, LANES)]
    w1 = wb_v[pl.ds(16, LANES)]
    w2 = wb_v[pl.ds(32, LANES)]
    w3 = wb_v[pl.ds(48, LANES)]
    bvec = wb_v[pl.ds(D, LANES)]
    lane = lax.iota(jnp.int32, LANES)
    last = jnp.full((LANES,), LANES - 1, jnp.int32)

    done = 0
    while done < B_PER_W:
        ch = min(CH, B_PER_W - done)
        cbase = done

        def fire_batch(j, carry, cbase=cbase):
            rb = cbase + j * K
            uvec = uidx_v[pl.ds(rb, K)]
            ivec = iidx_v[pl.ds(rb, K)]
            for r in range(K):
                pltpu.async_copy(ut_hbm.at[uvec[r]],
                                 urows_v.at[j * K + r], sem)
                pltpu.async_copy(it_hbm.at[ivec[r]],
                                 irows_v.at[j * K + r], sem)
            for r in range(K):
                pltpu.make_async_copy(ut_hbm.at[uvec[r]],
                                      urows_v.at[j * K + r], sem).wait()
                pltpu.make_async_copy(it_hbm.at[ivec[r]],
                                      irows_v.at[j * K + r], sem).wait()
            return carry

        lax.fori_loop(0, ch // K, fire_batch, 0, unroll=False)

        def group_body(g, carry, cbase=cbase):
            gbase = g * LANES
            for r in range(LANES):
                u = urows_v.at[gbase + r]
                it = irows_v.at[gbase + r]
                p = (u[pl.ds(0, LANES)] * it[pl.ds(0, LANES)] * w0
                     + u[pl.ds(16, LANES)] * it[pl.ds(16, LANES)] * w1
                     + u[pl.ds(32, LANES)] * it[pl.ds(32, LANES)] * w2
                     + u[pl.ds(48, LANES)] * it[pl.ds(48, LANES)] * w3)
                scan_v[r] = plsc.cumsum(p)
            sums = plsc.load_gather(scan_v, [lane, last])
            x = sums + bvec
            out_v[pl.ds(cbase + gbase, LANES)] = 1.0 / (1.0 + jnp.exp(-x))
            return carry

        lax.fori_loop(0, ch // LANES, group_body, 0, unroll=False)
        done += ch

    pltpu.sync_copy(out_v, out_hbm.at[pl.ds(base, B_PER_W)])


def _tc_body(uidx_s, iidx_s, b_s, ut_any, it_any, w_v, out_v, ubuf, ibuf, sem):
    w = w_v[...].reshape(1, D)
    b = b_s[0]

    def chunk(c, carry):
        cb = c * TCH

        def fire_row(r, carry2):
            pltpu.make_async_copy(ut_any.at[uidx_s[cb + r]],
                                  ubuf.at[r], sem).start()
            pltpu.make_async_copy(it_any.at[iidx_s[cb + r]],
                                  ibuf.at[r], sem).start()
            return carry2

        lax.fori_loop(0, TCH, fire_row, 0, unroll=8)
        pltpu.make_async_copy(ut_any.at[pl.ds(0, TCH)], ubuf, sem).wait()
        pltpu.make_async_copy(it_any.at[pl.ds(0, TCH)], ibuf, sem).wait()
        p = ubuf[...] * ibuf[...] * w
        s = jnp.sum(p, axis=1) + b
        out_v[pl.ds(cb, TCH)] = 1.0 / (1.0 + jnp.exp(-s))
        return carry

    lax.fori_loop(0, TC_ROWS // TCH, chunk, 0)


@jax.jit
def _gmf(user_indices, item_indices, user_table, item_table, W, wb, b):
    mesh = plsc.VectorSubcoreMesh(core_axis_name="c", subcore_axis_name="s")
    sc_run = pl.kernel(
        _sc_body,
        out_type=jax.ShapeDtypeStruct((SC_ROWS,), jnp.float32),
        mesh=mesh,
        compiler_params=pltpu.CompilerParams(
            needs_layout_passes=False, use_tc_tiling_on_sc=True),
        scratch_types=[
            pltpu.VMEM((B_PER_W,), jnp.int32),
            pltpu.VMEM((B_PER_W,), jnp.int32),
            pltpu.VMEM((CH, D), jnp.float32),
            pltpu.VMEM((CH, D), jnp.float32),
            pltpu.VMEM((D + LANES,), jnp.float32),
            pltpu.VMEM((B_PER_W,), jnp.float32),
            pltpu.VMEM((LANES, LANES), jnp.float32),
            pltpu.SemaphoreType.DMA,
        ],
    )
    out_sc = sc_run(user_table, item_table,
                    user_indices, item_indices, wb)

    out_tc = pl.pallas_call(
        _tc_body,
        out_shape=jax.ShapeDtypeStruct((TC_ROWS,), jnp.float32),
        in_specs=[
            pl.BlockSpec(memory_space=pltpu.SMEM),
            pl.BlockSpec(memory_space=pltpu.SMEM),
            pl.BlockSpec(memory_space=pltpu.SMEM),
            pl.BlockSpec(memory_space=pl.ANY),
            pl.BlockSpec(memory_space=pl.ANY),
            pl.BlockSpec(memory_space=pltpu.VMEM),
        ],
        out_specs=pl.BlockSpec(memory_space=pltpu.VMEM),
        scratch_shapes=[
            pltpu.VMEM((TCH, D), jnp.float32),
            pltpu.VMEM((TCH, D), jnp.float32),
            pltpu.SemaphoreType.DMA,
        ],
    )(user_indices[SC_ROWS:], item_indices[SC_ROWS:], b,
      user_table, item_table, W)

    return jnp.concatenate([out_sc, out_tc])


def kernel(user_indices, item_indices, user_table, item_table, W, b):
    wb = jnp.concatenate(
        [W.reshape(-1), jnp.broadcast_to(b.reshape(-1), (LANES,))])
    return _gmf(user_indices.astype(jnp.int32), item_indices.astype(jnp.int32),
                user_table, item_table, W.reshape(-1), wb, b)


# speedup vs baseline: 1.2366x; 1.0555x over previous
import jax
import jax.numpy as jnp
from jax import lax
from jax.experimental import pallas as pl
from jax.experimental.pallas import tpu as pltpu
from jax.experimental.pallas import tpu_sc as plsc

BATCH = 16384
D = 64
LANES = 16
NC = 2
NS = 16
NW = NC * NS
B_PER_W = BATCH // NW
CH = 256
K = 16


def _gmf_body(ut_hbm, it_hbm, uidx_hbm, iidx_hbm, wb_hbm, out_hbm,
              uidx_v, iidx_v, urows_v, irows_v, wb_v, out_v,
              scan_v, sem):
    wid = lax.axis_index("s") * NC + lax.axis_index("c")
    base = wid * B_PER_W

    pltpu.sync_copy(uidx_hbm.at[pl.ds(base, B_PER_W)], uidx_v)
    pltpu.sync_copy(iidx_hbm.at[pl.ds(base, B_PER_W)], iidx_v)
    pltpu.sync_copy(wb_hbm, wb_v)

    w0 = wb_v[pl.ds(0, LANES)]
    w1 = wb_v[pl.ds(16, LANES)]
    w2 = wb_v[pl.ds(32, LANES)]
    w3 = wb_v[pl.ds(48, LANES)]
    bvec = wb_v[pl.ds(D, LANES)]
    lane = lax.iota(jnp.int32, LANES)
    last = jnp.full((LANES,), LANES - 1, jnp.int32)

    for c in range(B_PER_W // CH):
        cbase = c * CH

        def fire_batch(j, carry):
            rb = cbase + j * K
            uvec = uidx_v[pl.ds(rb, K)]
            ivec = iidx_v[pl.ds(rb, K)]
            for r in range(K):
                pltpu.async_copy(ut_hbm.at[uvec[r]],
                                 urows_v.at[j * K + r], sem)
                pltpu.async_copy(it_hbm.at[ivec[r]],
                                 irows_v.at[j * K + r], sem)
            for r in range(K):
                pltpu.make_async_copy(ut_hbm.at[uvec[r]],
                                      urows_v.at[j * K + r], sem).wait()
                pltpu.make_async_copy(it_hbm.at[ivec[r]],
                                      irows_v.at[j * K + r], sem).wait()
            return carry

        lax.fori_loop(0, CH // K, fire_batch, 0, unroll=False)

        def group_body(g, carry):
            gbase = g * LANES
            for r in range(LANES):
                u = urows_v.at[gbase + r]
                it = irows_v.at[gbase + r]
                p = (u[pl.ds(0, LANES)] * it[pl.ds(0, LANES)] * w0
                     + u[pl.ds(16, LANES)] * it[pl.ds(16, LANES)] * w1
                     + u[pl.ds(32, LANES)] * it[pl.ds(32, LANES)] * w2
                     + u[pl.ds(48, LANES)] * it[pl.ds(48, LANES)] * w3)
                scan_v[r] = plsc.cumsum(p)
            sums = plsc.load_gather(scan_v, [lane, last])
            x = sums + bvec
            out_v[pl.ds(cbase + gbase, LANES)] = 1.0 / (1.0 + jnp.exp(-x))
            return carry

        lax.fori_loop(0, CH // LANES, group_body, 0, unroll=False)

    pltpu.sync_copy(out_v, out_hbm.at[pl.ds(base, B_PER_W)])


@jax.jit
def _gmf(user_indices, item_indices, user_table, item_table, wb):
    mesh = plsc.VectorSubcoreMesh(core_axis_name="c", subcore_axis_name="s")
    run = pl.kernel(
        _gmf_body,
        out_type=jax.ShapeDtypeStruct((BATCH,), jnp.float32),
        mesh=mesh,
        compiler_params=pltpu.CompilerParams(
            needs_layout_passes=False, use_tc_tiling_on_sc=True),
        scratch_types=[
            pltpu.VMEM((B_PER_W,), jnp.int32),
            pltpu.VMEM((B_PER_W,), jnp.int32),
            pltpu.VMEM((CH, D), jnp.float32),
            pltpu.VMEM((CH, D), jnp.float32),
            pltpu.VMEM((D + LANES,), jnp.float32),
            pltpu.VMEM((B_PER_W,), jnp.float32),
            pltpu.VMEM((LANES, LANES), jnp.float32),
            pltpu.SemaphoreType.DMA,
        ],
    )
    return run(user_table, item_table, user_indices, item_indices, wb)


def kernel(user_indices, item_indices, user_table, item_table, W, b):
    wb = jnp.concatenate(
        [W.reshape(-1), jnp.broadcast_to(b.reshape(-1), (LANES,))])
    return _gmf(user_indices.astype(jnp.int32), item_indices.astype(jnp.int32),
                user_table, item_table, wb)
